# Initial kernel scaffold; baseline (speedup 1.0000x reference)
import jax
import jax.numpy as jnp
from jax.experimental import pallas as pl


def kernel(features, edge_index, W1, b1, Wc1, bc1, Wc2, bc2, W2, b2):
    raise NotImplementedError("write your pallas kernel here")


# speedup vs baseline: 2.0806x; 2.0806x over previous
import functools

import jax
import jax.numpy as jnp
from jax import lax
from jax.experimental import pallas as pl
from jax.experimental.pallas import tpu as pltpu
from jax.experimental.pallas import tpu_sc as plsc

_N = 10000
_E = 160000
_F = 256
_H = 128

_NSC = 2
_NSUB = 16
_NW = _NSC * _NSUB
_BB = 128
_EP = 163840
_EPW = _EP // _NW
_NBATCH = _EPW // _BB
_ACC_ROWS = _N + 8
_RPT = 624
_RTAIL = _N - _NSUB * _RPT

_MB = 1000
_GRID = (_N // _MB,)


@functools.partial(
    pl.kernel,
    out_type=jax.ShapeDtypeStruct((_NSC, _N, _H), jnp.float32),
    mesh=plsc.VectorSubcoreMesh(core_axis_name="c", subcore_axis_name="s"),
    scratch_types=[
        pltpu.VMEM_SHARED((_ACC_ROWS, _H), jnp.float32),
        pltpu.VMEM((_BB,), jnp.int32),
        pltpu.VMEM((_BB,), jnp.int32),
        pltpu.VMEM((_BB, _H), jnp.float32),
        pltpu.SemaphoreType.DMA,
    ],
)
def _seg_sum_partials(h_hbm, src_hbm, dst_hbm, z_hbm, out_hbm,
                      acc, src_v, dst_v, rows_v, sem):
    c = lax.axis_index("c")
    s = lax.axis_index("s")
    pltpu.sync_copy(z_hbm, acc.at[pl.ds(s * _RPT, _RPT)])

    @pl.when(s == 0)
    def _zero_tail():
        pltpu.sync_copy(z_hbm.at[pl.ds(0, _RTAIL)],
                        acc.at[pl.ds(_NSUB * _RPT, _RTAIL)])

    plsc.subcore_barrier()
    ebase = (c * _NSUB + s) * _EPW

    def batch(b, carry):
        off = pl.multiple_of(ebase + b * _BB, _BB)
        pltpu.sync_copy(src_hbm.at[pl.ds(off, _BB)], src_v)
        pltpu.sync_copy(dst_hbm.at[pl.ds(off, _BB)], dst_v)
        pltpu.async_copy(h_hbm.at[src_v], rows_v, sem).wait()
        pltpu.sync_copy(rows_v, acc.at[dst_v], add=True)
        return carry

    lax.fori_loop(0, _NBATCH, batch, 0)
    plsc.subcore_barrier()
    pltpu.sync_copy(acc.at[pl.ds(s * _RPT, _RPT)],
                    out_hbm.at[c, pl.ds(s * _RPT, _RPT)])

    @pl.when(s == 0)
    def _flush_tail():
        pltpu.sync_copy(acc.at[pl.ds(_NSUB * _RPT, _RTAIL)],
                        out_hbm.at[c, pl.ds(_NSUB * _RPT, _RTAIL)])


def _dot(a, b):
    return jnp.dot(a, b, preferred_element_type=jnp.float32)


def _tc1(f_ref, w1_ref, b1_ref, wc1_ref, x_ref, h1_ref):
    x = jnp.maximum(_dot(f_ref[...], w1_ref[...]) + b1_ref[...], 0.0)
    x_ref[...] = x
    h1_ref[...] = _dot(x, wc1_ref[...])


def _tc2(p_ref, bc1_ref, wc1_ref, x11_ref, h2_ref):
    x11 = p_ref[0] + p_ref[1] + bc1_ref[...]
    x11_ref[...] = x11
    h2_ref[...] = _dot(x11, wc1_ref[...])


def _tc3(p_ref, bc1_ref, x11_ref, waa_ref, wab_ref, wba_ref, wbb_ref,
         x12_ref, h3lo_ref, h3hi_ref):
    x12 = p_ref[0] + p_ref[1] + bc1_ref[...]
    x12_ref[...] = x12
    x11 = x11_ref[...]
    h3lo_ref[...] = _dot(x11, waa_ref[...]) + _dot(x12, wba_ref[...])
    h3hi_ref[...] = _dot(x11, wab_ref[...]) + _dot(x12, wbb_ref[...])


def _tc4(plo_ref, phi_ref, bc2lo_ref, bc2hi_ref,
         waa_ref, wab_ref, wba_ref, wbb_ref,
         x21lo_ref, x21hi_ref, h4lo_ref, h4hi_ref):
    x21lo = plo_ref[0] + plo_ref[1] + bc2lo_ref[...]
    x21hi = phi_ref[0] + phi_ref[1] + bc2hi_ref[...]
    x21lo_ref[...] = x21lo
    x21hi_ref[...] = x21hi
    h4lo_ref[...] = _dot(x21lo, waa_ref[...]) + _dot(x21hi, wba_ref[...])
    h4hi_ref[...] = _dot(x21lo, wab_ref[...]) + _dot(x21hi, wbb_ref[...])


def _tc5(plo_ref, phi_ref, bc2lo_ref, bc2hi_ref,
         x_ref, x11_ref, x12_ref, x21lo_ref, x21hi_ref,
         w0_ref, w1_ref, w2_ref, w3_ref, w4_ref, w5_ref, w6_ref, b2_ref,
         out_ref):
    x22lo = plo_ref[0] + plo_ref[1] + bc2lo_ref[...]
    x22hi = phi_ref[0] + phi_ref[1] + bc2hi_ref[...]
    acc = (_dot(x_ref[...], w0_ref[...]) + _dot(x11_ref[...], w1_ref[...])
           + _dot(x12_ref[...], w2_ref[...]) + _dot(x21lo_ref[...], w3_ref[...])
           + _dot(x21hi_ref[...], w4_ref[...]) + _dot(x22lo, w5_ref[...])
           + _dot(x22hi, w6_ref[...]) + b2_ref[...])
    out_ref[...] = jax.nn.sigmoid(acc)


def _rows(k):
    return pl.BlockSpec((_MB, k), lambda i: (i, 0))


def _full(r, k):
    return pl.BlockSpec((r, k), lambda i: (0, 0))


def _part():
    return pl.BlockSpec((_NSC, _MB, _H), lambda i: (0, i, 0))


def _mshape(k=_H):
    return jax.ShapeDtypeStruct((_N, k), jnp.float32)


def kernel(features, edge_index, W1, b1, Wc1, bc1, Wc2, bc2, W2, b2):
    src = edge_index[0].astype(jnp.int32)
    dst = edge_index[1].astype(jnp.int32)
    npad = _EP - _E
    srcp = jnp.concatenate([src, jnp.zeros((npad,), jnp.int32)])
    dstp = jnp.concatenate([dst, jnp.full((npad,), _N, jnp.int32)])
    zrows = jnp.zeros((_RPT, _H), jnp.float32)

    b1r = b1.reshape(1, _H)
    bc1r = bc1.reshape(1, _H)
    bc2lo = bc2[:_H].reshape(1, _H)
    bc2hi = bc2[_H:].reshape(1, _H)
    b2r = b2.reshape(1, -1)
    waa, wab = Wc2[:_H, :_H], Wc2[:_H, _H:]
    wba, wbb = Wc2[_H:, :_H], Wc2[_H:, _H:]
    w2p = [W2[k * _H:(k + 1) * _H] for k in range(7)]
    c_out = W2.shape[1]

    seg = lambda h: _seg_sum_partials(h, srcp, dstp, zrows)

    x, h1 = pl.pallas_call(
        _tc1, grid=_GRID,
        in_specs=[_rows(_F), _full(_F, _H), _full(1, _H), _full(_H, _H)],
        out_specs=[_rows(_H), _rows(_H)],
        out_shape=[_mshape(), _mshape()],
    )(features, W1, b1r, Wc1)

    p1 = seg(h1)
    x11, h2 = pl.pallas_call(
        _tc2, grid=_GRID,
        in_specs=[_part(), _full(1, _H), _full(_H, _H)],
        out_specs=[_rows(_H), _rows(_H)],
        out_shape=[_mshape(), _mshape()],
    )(p1, bc1r, Wc1)

    p2 = seg(h2)
    x12, h3lo, h3hi = pl.pallas_call(
        _tc3, grid=_GRID,
        in_specs=[_part(), _full(1, _H), _rows(_H)] + [_full(_H, _H)] * 4,
        out_specs=[_rows(_H)] * 3,
        out_shape=[_mshape()] * 3,
    )(p2, bc1r, x11, waa, wab, wba, wbb)

    p3lo = seg(h3lo)
    p3hi = seg(h3hi)
    x21lo, x21hi, h4lo, h4hi = pl.pallas_call(
        _tc4, grid=_GRID,
        in_specs=[_part(), _part(), _full(1, _H), _full(1, _H)]
                 + [_full(_H, _H)] * 4,
        out_specs=[_rows(_H)] * 4,
        out_shape=[_mshape()] * 4,
    )(p3lo, p3hi, bc2lo, bc2hi, waa, wab, wba, wbb)

    p4lo = seg(h4lo)
    p4hi = seg(h4hi)
    out = pl.pallas_call(
        _tc5, grid=_GRID,
        in_specs=[_part(), _part(), _full(1, _H), _full(1, _H)]
                 + [_rows(_H)] * 5 + [_full(_H, c_out)] * 7
                 + [_full(1, c_out)],
        out_specs=pl.BlockSpec((_MB, c_out), lambda i: (i, 0)),
        out_shape=jax.ShapeDtypeStruct((_N, c_out), jnp.float32),
    )(p4lo, p4hi, bc2lo, bc2hi, x, x11, x12, x21lo, x21hi, *w2p, b2r)

    return out
